# baseline (speedup 1.0000x reference)
import functools

import jax
import jax.numpy as jnp
from jax import lax
from jax.experimental import pallas as pl
from jax.experimental.pallas import tpu as pltpu
from jax.experimental.pallas import tpu_sc as plsc

N = 16384
L = 16
D = 4
G = L * D
NB = 64
NC = 2
NS = 16
NW = NC * NS
GPT = G // NW
LANES = 16
GS = NB + 1
SUBH = LANES * GS
CWORDS = GPT * SUBH

_mesh = plsc.VectorSubcoreMesh(
    core_axis_name="c", subcore_axis_name="s", num_cores=NC, num_subcores=NS)
_params = pltpu.CompilerParams(
    needs_layout_passes=False, use_tc_tiling_on_sc=False)

_ACC = 8


@functools.partial(
    pl.kernel,
    out_type=jax.ShapeDtypeStruct((NW, LANES), jnp.float32),
    mesh=_mesh,
    compiler_params=_params,
    scratch_types=[pltpu.VMEM((GPT, 128, 128), jnp.float32),
                   pltpu.VMEM((GPT, 128, 128), jnp.float32),
                   pltpu.VMEM((CWORDS,), jnp.float32),
                   pltpu.VMEM((CWORDS,), jnp.float32),
                   pltpu.VMEM((LANES,), jnp.float32),
                   pltpu.SemaphoreType.DMA,
                   pltpu.SemaphoreType.DMA,
                   pltpu.SemaphoreType.DMA,
                   pltpu.SemaphoreType.DMA],
)
def _sc_hist_loss(xr_hbm, xf_hbm, out_hbm, rbuf, fbuf, cr, cf, obuf,
                  rsem0, fsem0, rsem1, fsem1):
    wid = lax.axis_index("s") * NC + lax.axis_index("c")
    l0 = (GPT * wid) // D
    d0 = (GPT * wid) % D
    l1 = (GPT * wid + 1) // D
    d1 = (GPT * wid + 1) % D
    rcp0 = pltpu.async_copy(xr_hbm.at[l0, :, d0, :], rbuf.at[0], rsem0)
    rcp1 = pltpu.async_copy(xr_hbm.at[l1, :, d1, :], rbuf.at[1], rsem1)
    fcp0 = pltpu.async_copy(xf_hbm.at[l0, :, d0, :], fbuf.at[0], fsem0)
    fcp1 = pltpu.async_copy(xf_hbm.at[l1, :, d1, :], fbuf.at[1], fsem1)

    zeros = jnp.zeros((LANES,), jnp.float32)

    @plsc.parallel_loop(0, CWORDS // LANES, unroll=5)
    def _(i):
        cr[pl.ds(i * LANES, LANES)] = zeros
        cf[pl.ds(i * LANES, LANES)] = zeros

    lane = jnp.arange(LANES, dtype=jnp.int32)
    ones = jnp.ones((LANES,), jnp.float32)
    rwaits = [rcp0.wait, rcp1.wait]
    fwaits = [fcp0.wait, fcp1.wait]
    params = []
    for g in range(GPT):
        rwaits[g]()
        first = [rbuf[g, 0, pl.ds(k * LANES, LANES)] for k in range(_ACC)]

        def mbody(i, carry, g=g):
            out_mn, out_mx = [], []
            for k in range(_ACC):
                j = i * _ACC + k
                x = rbuf[g, j // 8, pl.ds((j % 8) * LANES, LANES)]
                out_mn.append(jnp.minimum(carry[k], x))
                out_mx.append(jnp.maximum(carry[_ACC + k], x))
            return tuple(out_mn + out_mx)

        red = lax.fori_loop(1, N // LANES // _ACC, mbody, tuple(first) * 2)
        mn_v = red[0]
        mx_v = red[_ACC]
        for k in range(1, _ACC):
            mn_v = jnp.minimum(mn_v, red[k])
            mx_v = jnp.maximum(mx_v, red[_ACC + k])
        mn = jnp.min(mn_v)
        mx = jnp.max(mx_v)
        degen = jnp.abs(mx - mn) < 1e-10
        mx = jnp.where(degen, mx + 1e-05, mx)
        mn = jnp.where(degen, mn - 1e-05, mn)
        delta = (mx - mn) * (1.0 / NB)
        mnb = jnp.full((LANES,), mn, jnp.float32)
        deltab = jnp.full((LANES,), delta, jnp.float32)
        invdb = 1.0 / deltab
        halfwb = deltab * 0.5
        params.append((mnb, deltab, invdb, halfwb, delta))

    losses = []
    for g in range(GPT):
        mnb, deltab, invdb, halfwb, delta = params[g]
        cbase = g * SUBH + lane * GS
        fwaits[g]()

        @plsc.parallel_loop(0, N // LANES, unroll=8)
        def _(i, g=g, mnb=mnb, deltab=deltab, invdb=invdb,
              halfwb=halfwb, cbase=cbase):
            xr_v = rbuf[g, i // 8, pl.ds((i % 8) * LANES, LANES)]
            tr = (xr_v - mnb) * invdb
            ir = jnp.minimum(tr.astype(jnp.int32), NB - 1)
            plsc.addupdate_scatter(cr, [cbase + ir], ones)
            xf_v = fbuf[g, i // 8, pl.ds((i % 8) * LANES, LANES)]
            tf = (xf_v - mnb) * invdb
            jf = tf.astype(jnp.int32)
            jf = jnp.minimum(jnp.maximum(jf, 0), NB - 1)
            center = mnb + deltab * (jf.astype(jnp.float32) + 0.5)
            hit = halfwb > jnp.abs(xf_v - center)
            plsc.addupdate_scatter(cf, [cbase + jf], ones, mask=hit)

        nj = NB // LANES

        def fbody(s, carry, g=g):
            out = []
            for j in range(nj):
                off = g * SUBH + s * GS + j * LANES
                out.append(carry[j] + cr[pl.ds(off, LANES)])
            for j in range(nj):
                off = g * SUBH + s * GS + j * LANES
                out.append(carry[nj + j] + cf[pl.ds(off, LANES)])
            return tuple(out)

        acc = lax.fori_loop(0, LANES, fbody,
                            (jnp.zeros((LANES,), jnp.float32),) * (2 * nj))
        svec = jnp.abs(acc[nj] - acc[0])
        for j in range(1, nj):
            svec = svec + jnp.abs(acc[nj + j] - acc[j])
        losses.append((jnp.sum(svec), delta))

    lv = jnp.where(lane == 0, losses[0][0], losses[1][0])
    dv = jnp.where(lane == 0, losses[0][1], losses[1][1])
    obuf[...] = lv / (dv * float(NB * N))
    pltpu.sync_copy(obuf, out_hbm.at[wid])


def _as_tiled_view(x):
    return x.transpose(1, 0, 2).reshape(L, 128, 128, D).transpose(0, 1, 3, 2)


def kernel(x_fake, x_real):
    out = _sc_hist_loss(_as_tiled_view(x_real), _as_tiled_view(x_fake))
    return out[:, :GPT].reshape(L, D)


# speedup vs baseline: 1.0563x; 1.0563x over previous
import functools

import jax
import jax.numpy as jnp
from jax import lax
from jax.experimental import pallas as pl
from jax.experimental.pallas import tpu as pltpu
from jax.experimental.pallas import tpu_sc as plsc

N = 16384
L = 16
D = 4
G = L * D
NB = 64
NC = 2
NS = 16
NW = NC * NS
GPT = G // NW
LANES = 16
GS = NB + 1
SUBH = LANES * GS
CWORDS = GPT * SUBH

_mesh = plsc.VectorSubcoreMesh(
    core_axis_name="c", subcore_axis_name="s", num_cores=NC, num_subcores=NS)
_params = pltpu.CompilerParams(
    needs_layout_passes=False, use_tc_tiling_on_sc=False)

_ACC = 8


@functools.partial(
    pl.kernel,
    out_type=jax.ShapeDtypeStruct((NW, LANES), jnp.float32),
    mesh=_mesh,
    compiler_params=_params,
    scratch_types=[pltpu.VMEM((GPT, 128, 128), jnp.float32),
                   pltpu.VMEM((GPT, 128, 128), jnp.float32),
                   pltpu.VMEM((CWORDS,), jnp.float32),
                   pltpu.VMEM((CWORDS,), jnp.float32),
                   pltpu.VMEM((LANES,), jnp.float32),
                   pltpu.SemaphoreType.DMA,
                   pltpu.SemaphoreType.DMA,
                   pltpu.SemaphoreType.DMA,
                   pltpu.SemaphoreType.DMA],
)
def _sc_hist_loss(xr_hbm, xf_hbm, out_hbm, rbuf, fbuf, cr, cf, obuf,
                  rsem0, fsem0, rsem1, fsem1):
    wid = lax.axis_index("s") * NC + lax.axis_index("c")
    l0 = (GPT * wid) // D
    d0 = (GPT * wid) % D
    l1 = (GPT * wid + 1) // D
    d1 = (GPT * wid + 1) % D
    rcp0 = pltpu.async_copy(xr_hbm.at[l0, :, d0, :], rbuf.at[0], rsem0)
    rcp1 = pltpu.async_copy(xr_hbm.at[l1, :, d1, :], rbuf.at[1], rsem1)
    fcp0 = pltpu.async_copy(xf_hbm.at[l0, :, d0, :], fbuf.at[0], fsem0)
    fcp1 = pltpu.async_copy(xf_hbm.at[l1, :, d1, :], fbuf.at[1], fsem1)

    zeros = jnp.zeros((LANES,), jnp.float32)

    @plsc.parallel_loop(0, CWORDS // LANES, unroll=5)
    def _(i):
        cr[pl.ds(i * LANES, LANES)] = zeros
        cf[pl.ds(i * LANES, LANES)] = zeros

    lane = jnp.arange(LANES, dtype=jnp.int32)
    ones = jnp.ones((LANES,), jnp.float32)
    rwaits = [rcp0.wait, rcp1.wait]
    fwaits = [fcp0.wait, fcp1.wait]
    params = []
    for g in range(GPT):
        rwaits[g]()
        first = [rbuf[g, 0, pl.ds(k * LANES, LANES)] for k in range(_ACC)]

        def mbody(i, carry, g=g):
            out_mn, out_mx = [], []
            for k in range(_ACC):
                j = i * _ACC + k
                x = rbuf[g, j // 8, pl.ds((j % 8) * LANES, LANES)]
                out_mn.append(jnp.minimum(carry[k], x))
                out_mx.append(jnp.maximum(carry[_ACC + k], x))
            return tuple(out_mn + out_mx)

        red = lax.fori_loop(1, N // LANES // _ACC, mbody, tuple(first) * 2)
        mn_v = red[0]
        mx_v = red[_ACC]
        for k in range(1, _ACC):
            mn_v = jnp.minimum(mn_v, red[k])
            mx_v = jnp.maximum(mx_v, red[_ACC + k])
        mn = jnp.min(mn_v)
        mx = jnp.max(mx_v)
        degen = jnp.abs(mx - mn) < 1e-10
        mx = jnp.where(degen, mx + 1e-05, mx)
        mn = jnp.where(degen, mn - 1e-05, mn)
        delta = (mx - mn) * (1.0 / NB)
        mnb = jnp.full((LANES,), mn, jnp.float32)
        deltab = jnp.full((LANES,), delta, jnp.float32)
        invdb = 1.0 / deltab
        halfwb = deltab * 0.5
        params.append((mnb, deltab, invdb, halfwb, delta))

    losses = []
    for g in range(GPT):
        mnb, deltab, invdb, halfwb, delta = params[g]
        cbase = g * SUBH + lane * GS
        fwaits[g]()

        @plsc.parallel_loop(0, N // LANES, unroll=8)
        def _(i, g=g, mnb=mnb, deltab=deltab, invdb=invdb,
              halfwb=halfwb, cbase=cbase):
            xr_v = rbuf[g, i // 8, pl.ds((i % 8) * LANES, LANES)]
            tr = (xr_v - mnb) * invdb
            ir = tr.astype(jnp.int32)
            plsc.addupdate_scatter(cr, [cbase + ir], ones)
            xf_v = fbuf[g, i // 8, pl.ds((i % 8) * LANES, LANES)]
            tf = (xf_v - mnb) * invdb
            jf = tf.astype(jnp.int32)
            jf = jnp.minimum(jnp.maximum(jf, 0), NB)
            jff = jf.astype(jnp.float32)
            hit = (tf > jff) & (tf < jff + 1.0)
            plsc.addupdate_scatter(cf, [cbase + jf], ones, mask=hit)

        nj = NB // LANES

        def fbody(s, carry, g=g):
            out = []
            for j in range(nj):
                off = g * SUBH + s * GS + j * LANES
                out.append(carry[j] + cr[pl.ds(off, LANES)])
            for j in range(nj):
                off = g * SUBH + s * GS + j * LANES
                out.append(carry[nj + j] + cf[pl.ds(off, LANES)])
            return tuple(out)

        acc = lax.fori_loop(0, LANES, fbody,
                            (jnp.zeros((LANES,), jnp.float32),) * (2 * nj))
        pads = plsc.load_gather(cr, [g * SUBH + lane * GS + NB])
        accr_last = acc[nj - 1] + jnp.sum(pads) * (lane == LANES - 1)
        svec = jnp.abs(acc[2 * nj - 1] - accr_last)
        for j in range(nj - 1):
            svec = svec + jnp.abs(acc[nj + j] - acc[j])
        losses.append((jnp.sum(svec), delta))

    lv = jnp.where(lane == 0, losses[0][0], losses[1][0])
    dv = jnp.where(lane == 0, losses[0][1], losses[1][1])
    obuf[...] = lv / (dv * float(NB * N))
    pltpu.sync_copy(obuf, out_hbm.at[wid])


def _as_tiled_view(x):
    return x.transpose(1, 0, 2).reshape(L, 128, 128, D).transpose(0, 1, 3, 2)


def kernel(x_fake, x_real):
    out = _sc_hist_loss(_as_tiled_view(x_real), _as_tiled_view(x_fake))
    return out[:, :GPT].reshape(L, D)
